# baseline (speedup 1.0000x reference)
import functools
import math

import jax
import jax.numpy as jnp
from jax import lax
from jax.experimental import pallas as pl
from jax.experimental.pallas import tpu as pltpu
from jax.experimental.pallas import tpu_sc as plsc

H = 64
SCALE = float(math.sqrt(H))
NW = 32
CB = 2048
V = 100000


def _make_encoder(B, L):
    assert B % 128 == 0 and B // CB in (1, 2) and CB % 128 == 0
    halves = B // CB
    nq = L * halves
    assert nq % 2 == 0
    mesh = plsc.VectorSubcoreMesh(core_axis_name="c", subcore_axis_name="s")

    @functools.partial(
        pl.kernel,
        mesh=mesh,
        compiler_params=pltpu.CompilerParams(use_tc_tiling_on_sc=False,
                                             needs_layout_passes=False),
        out_type=jax.ShapeDtypeStruct((L, H // 8, B // 128, 8, 128),
                                      jnp.float32),
        scratch_types=[
            pltpu.VMEM((V,), jnp.float32),
            pltpu.VMEM((128,), jnp.float32),
            pltpu.VMEM((128,), jnp.float32),
            pltpu.VMEM((2, CB), jnp.int32),
            pltpu.VMEM((2, CB), jnp.float32),
            pltpu.VMEM((2, 2, CB // 128, 1, 128), jnp.float32),
            pltpu.SemaphoreType.DMA((2,)),
            pltpu.SemaphoreType.DMA((2,)),
        ],
    )
    def enc(pkT_hbm, durT_hbm, epkT_hbm, sm_hbm,
            out_hbm, ecol_v, sma_v, smb_v, tok_v, dur_v, ob_v, isem, osem):
        wid = lax.axis_index("s") * 2 + lax.axis_index("c")
        th = wid // 8
        hr = wid % 8

        def issue(q, b):
            l = q // halves
            b0 = (q % halves) * CB
            pltpu.async_copy(pkT_hbm.at[l, pl.ds(b0, CB)], tok_v.at[b],
                             isem.at[b])
            pltpu.async_copy(durT_hbm.at[l, pl.ds(b0, CB)], dur_v.at[b],
                             isem.at[b])

        def drain_in(b):
            pltpu.make_async_copy(pkT_hbm.at[0, pl.ds(0, CB)], tok_v.at[b],
                                  isem.at[b]).wait()
            pltpu.make_async_copy(durT_hbm.at[0, pl.ds(0, CB)], dur_v.at[b],
                                  isem.at[b]).wait()

        def wait_out(b):
            for p in range(2):
                pltpu.make_async_copy(ob_v.at[b, p],
                                      out_hbm.at[0, 0, pl.ds(0, CB // 128),
                                                 pl.ds(0, 1), :],
                                      osem.at[b]).wait()

        pltpu.sync_copy(epkT_hbm.at[wid], ecol_v)
        pltpu.sync_copy(sm_hbm.at[wid], sma_v)
        pltpu.sync_copy(sm_hbm.at[wid + 32], smb_v)
        dwa = sma_v[pl.ds(5 * 16, 16)]
        dwb = smb_v[pl.ds(5 * 16, 16)]
        iota16 = lax.iota(jnp.int32, 16)
        himask = jnp.full((16,), 0xFFFF0000, jnp.uint32)

        def compute(q, b):
            @plsc.parallel_loop(0, CB // 16, unroll=32)
            def grp(gi):
                sl = pl.ds(gi * 16, 16)
                pk16 = tok_v[b, sl]
                d16 = dur_v[b, sl]
                tok16 = lax.shift_right_logical(pk16, 7)
                t2i = (pk16 & 127) + iota16
                e16 = plsc.load_gather(ecol_v, [tok16])
                ta = plsc.load_gather(sma_v, [t2i])
                tb = plsc.load_gather(smb_v, [t2i])
                u = plsc.bitcast(e16, jnp.uint32)
                ea = plsc.bitcast(lax.shift_left(u, jnp.uint32(16)),
                                  jnp.float32)
                eb = plsc.bitcast(u & himask, jnp.float32)
                osl = pl.ds((gi % 8) * 16, 16)
                ob_v[b, 0, gi // 8, 0, osl] = ea + (d16 * dwa + ta)
                ob_v[b, 1, gi // 8, 0, osl] = eb + (d16 * dwb + tb)

            l = q // halves
            tb0 = (q % halves) * (CB // 128)
            pltpu.async_copy(ob_v.at[b, 0],
                             out_hbm.at[l, th, pl.ds(tb0, CB // 128),
                                        pl.ds(hr, 1), :],
                             osem.at[b])
            pltpu.async_copy(ob_v.at[b, 1],
                             out_hbm.at[l, th + 4, pl.ds(tb0, CB // 128),
                                        pl.ds(hr, 1), :],
                             osem.at[b])

        issue(0, 0)

        def pair(it, carry):
            q0 = it * 2
            issue(q0 + 1, 1)

            @pl.when(it >= 1)
            def _():
                wait_out(0)
            drain_in(0)
            compute(q0, 0)

            @pl.when(q0 + 2 < nq)
            def _():
                issue(q0 + 2, 0)

            @pl.when(it >= 1)
            def _():
                wait_out(1)
            drain_in(1)
            compute(q0 + 1, 1)
            return carry

        lax.fori_loop(0, nq // 2, pair, 0)
        wait_out(0)
        wait_out(1)

    return enc


def kernel(note_tokens, note_durs, note_types, emb_weight, type_emb_weight,
           dur_w, dur_b):
    B, L = note_tokens.shape
    enc = _make_encoder(B, L)
    pk_t = (note_tokens.astype(jnp.int32) * 128
            + note_types.astype(jnp.int32) * 16).T
    dur_t = note_durs.T
    ebf = lax.bitcast_convert_type(
        (emb_weight * SCALE).astype(jnp.bfloat16), jnp.uint16)
    epk = lax.bitcast_convert_type(
        ebf[:, :32].astype(jnp.uint32)
        | (ebf[:, 32:].astype(jnp.uint32) << 16), jnp.float32)
    epk_t = epk.T
    t2_t = (type_emb_weight * SCALE + dur_b[None, :]).T
    sm = jnp.repeat(
        jnp.concatenate(
            [t2_t, dur_w[:, None], jnp.zeros((H, 2), jnp.float32)], axis=1),
        16, axis=1)
    out5 = enc(pk_t, dur_t, epk_t, sm)
    return out5.transpose(2, 4, 0, 1, 3).reshape(B, L, H)


# speedup vs baseline: 1.4297x; 1.4297x over previous
import functools
import math

import jax
import jax.numpy as jnp
from jax import lax
from jax.experimental import pallas as pl
from jax.experimental.pallas import tpu as pltpu
from jax.experimental.pallas import tpu_sc as plsc

H = 64
SCALE = float(math.sqrt(H))
NW = 32
CB = 2048
V = 100000


def _make_encoder(B, L):
    assert B % 128 == 0 and B // CB in (1, 2) and CB % 128 == 0
    halves = B // CB
    nq = L * halves
    assert nq % 2 == 0
    mesh = plsc.VectorSubcoreMesh(core_axis_name="c", subcore_axis_name="s")

    @functools.partial(
        pl.kernel,
        mesh=mesh,
        compiler_params=pltpu.CompilerParams(use_tc_tiling_on_sc=False,
                                             needs_layout_passes=False),
        out_type=jax.ShapeDtypeStruct((L, H // 8, B // 128, 8, 128),
                                      jnp.float32),
        scratch_types=[
            pltpu.VMEM((V,), jnp.float32),
            pltpu.VMEM((128,), jnp.float32),
            pltpu.VMEM((128,), jnp.float32),
            pltpu.VMEM((2, CB), jnp.int32),
            pltpu.VMEM((2, CB), jnp.float32),
            pltpu.VMEM((2, 2, CB // 128, 1, 128), jnp.float32),
            pltpu.SemaphoreType.DMA((2,)),
            pltpu.SemaphoreType.DMA((2,)),
        ],
    )
    def enc(pkT_hbm, durT_hbm, epkT_hbm, sm_hbm,
            out_hbm, ecol_v, sma_v, smb_v, tok_v, dur_v, ob_v, isem, osem):
        wid = lax.axis_index("s") * 2 + lax.axis_index("c")
        th = wid // 8
        hr = wid % 8

        def issue(q, b):
            l = q // halves
            b0 = (q % halves) * CB
            pltpu.async_copy(pkT_hbm.at[l, pl.ds(b0, CB)], tok_v.at[b],
                             isem.at[b])
            pltpu.async_copy(durT_hbm.at[l, pl.ds(b0, CB)], dur_v.at[b],
                             isem.at[b])

        def drain_in(b):
            pltpu.make_async_copy(pkT_hbm.at[0, pl.ds(0, CB)], tok_v.at[b],
                                  isem.at[b]).wait()
            pltpu.make_async_copy(durT_hbm.at[0, pl.ds(0, CB)], dur_v.at[b],
                                  isem.at[b]).wait()

        def wait_out(b):
            for p in range(2):
                pltpu.make_async_copy(ob_v.at[b, p],
                                      out_hbm.at[0, 0, pl.ds(0, CB // 128),
                                                 pl.ds(0, 1), :],
                                      osem.at[b]).wait()

        pltpu.sync_copy(epkT_hbm.at[wid], ecol_v)
        pltpu.sync_copy(sm_hbm.at[wid], sma_v)
        pltpu.sync_copy(sm_hbm.at[wid + 32], smb_v)
        dwa = sma_v[pl.ds(5 * 16, 16)]
        dwb = smb_v[pl.ds(5 * 16, 16)]
        iota16 = lax.iota(jnp.int32, 16)
        himask = jnp.full((16,), 0xFFFF0000, jnp.uint32)

        def compute(q, b):
            @plsc.parallel_loop(0, CB // 16, unroll=8)
            def grp(gi):
                sl = pl.ds(gi * 16, 16)
                pk16 = tok_v[b, sl]
                d16 = dur_v[b, sl]
                tok16 = lax.shift_right_logical(pk16, 7)
                t2i = (pk16 & 127) + iota16
                e16 = plsc.load_gather(ecol_v, [tok16])
                ta = plsc.load_gather(sma_v, [t2i])
                tb = plsc.load_gather(smb_v, [t2i])
                u = plsc.bitcast(e16, jnp.uint32)
                ea = plsc.bitcast(lax.shift_left(u, jnp.uint32(16)),
                                  jnp.float32)
                eb = plsc.bitcast(u & himask, jnp.float32)
                osl = pl.ds((gi % 8) * 16, 16)
                ob_v[b, 0, gi // 8, 0, osl] = ea + (d16 * dwa + ta)
                ob_v[b, 1, gi // 8, 0, osl] = eb + (d16 * dwb + tb)

            l = q // halves
            tb0 = (q % halves) * (CB // 128)
            pltpu.async_copy(ob_v.at[b, 0],
                             out_hbm.at[l, th, pl.ds(tb0, CB // 128),
                                        pl.ds(hr, 1), :],
                             osem.at[b])
            pltpu.async_copy(ob_v.at[b, 1],
                             out_hbm.at[l, th + 4, pl.ds(tb0, CB // 128),
                                        pl.ds(hr, 1), :],
                             osem.at[b])

        issue(0, 0)

        def pair(it, carry):
            q0 = it * 2
            issue(q0 + 1, 1)

            @pl.when(it >= 1)
            def _():
                wait_out(0)
            drain_in(0)
            compute(q0, 0)

            @pl.when(q0 + 2 < nq)
            def _():
                issue(q0 + 2, 0)

            @pl.when(it >= 1)
            def _():
                wait_out(1)
            drain_in(1)
            compute(q0 + 1, 1)
            return carry

        lax.fori_loop(0, nq // 2, pair, 0)
        wait_out(0)
        wait_out(1)

    return enc


def kernel(note_tokens, note_durs, note_types, emb_weight, type_emb_weight,
           dur_w, dur_b):
    B, L = note_tokens.shape
    enc = _make_encoder(B, L)
    pk_t = (note_tokens.astype(jnp.int32) * 128
            + note_types.astype(jnp.int32) * 16).T
    dur_t = note_durs.T
    ebf = lax.bitcast_convert_type(
        (emb_weight * SCALE).astype(jnp.bfloat16), jnp.uint16)
    epk = lax.bitcast_convert_type(
        ebf[:, :32].astype(jnp.uint32)
        | (ebf[:, 32:].astype(jnp.uint32) << 16), jnp.float32)
    epk_t = epk.T
    t2_t = (type_emb_weight * SCALE + dur_b[None, :]).T
    sm = jnp.repeat(
        jnp.concatenate(
            [t2_t, dur_w[:, None], jnp.zeros((H, 2), jnp.float32)], axis=1),
        16, axis=1)
    out5 = enc(pk_t, dur_t, epk_t, sm)
    return out5.transpose(2, 4, 0, 1, 3).reshape(B, L, H)
